# speedup vs baseline: 1.2303x; 1.2303x over previous
import functools

import jax
import jax.numpy as jnp
from jax import lax
from jax.experimental import pallas as pl
from jax.experimental.pallas import tpu as pltpu
from jax.experimental.pallas import tpu_sc as plsc

N_NODES = 100000
D_FEAT = 128
CHUNK = 80
NCHUNK = N_NODES // CHUNK

_INFO = plsc.get_sparse_core_info()
_NC = _INFO.num_cores
_NS = _INFO.num_subcores
_NW = _NC * _NS
_BASE = NCHUNK // _NW
_EXTRA = NCHUNK - _BASE * _NW


def _sc_body(x_hbm, idx_hbm, tab_hbm, out_hbm, idx_v, rows_v, x_v,
             sem_g, sem_x):
    wid = lax.axis_index("s") * _NC + lax.axis_index("c")
    start = wid * _BASE + jnp.minimum(wid, _EXTRA)
    count = _BASE + jnp.where(wid < _EXTRA, 1, 0)

    def chunk_body(j, carry):
        base = (start + j) * CHUNK
        pltpu.sync_copy(idx_hbm.at[pl.ds(base, CHUNK)], idx_v)
        cp_g = pltpu.async_copy(tab_hbm.at[idx_v], rows_v, sem_g)
        cp_x = pltpu.async_copy(x_hbm.at[pl.ds(base, CHUNK), :], x_v, sem_x)
        cp_g.wait()
        cp_x.wait()

        def row_body(r, c2):
            for c in range(D_FEAT // 16):
                sl = pl.ds(c * 16, 16)
                plsc.addupdate(x_v.at[r, sl], rows_v[r, sl])
            return c2

        lax.fori_loop(0, CHUNK, row_body, 0, unroll=False)
        pltpu.sync_copy(x_v, out_hbm.at[pl.ds(base, CHUNK), :])
        return carry

    lax.fori_loop(0, count, chunk_body, 0, unroll=False)


@jax.jit
def _run(x, idx, tab):
    mesh = plsc.VectorSubcoreMesh(core_axis_name="c", subcore_axis_name="s")
    f = pl.kernel(
        _sc_body,
        out_type=jax.ShapeDtypeStruct((N_NODES, D_FEAT), jnp.float32),
        mesh=mesh,
        scratch_types=[
            pltpu.VMEM((CHUNK,), jnp.int32),
            pltpu.VMEM((CHUNK, D_FEAT), jnp.float32),
            pltpu.VMEM((CHUNK, D_FEAT), jnp.float32),
            pltpu.SemaphoreType.DMA,
            pltpu.SemaphoreType.DMA,
        ],
    )
    return f(x, idx, tab)


def kernel(x, node_types, type_table):
    idx = node_types.astype(jnp.int32)
    return _run(x, idx, type_table)


# speedup vs baseline: 1.2412x; 1.0088x over previous
import functools

import jax
import jax.numpy as jnp
from jax import lax
from jax.experimental import pallas as pl
from jax.experimental.pallas import tpu as pltpu
from jax.experimental.pallas import tpu_sc as plsc

N_NODES = 100000
D_FEAT = 128
CHUNK = 80
NCHUNK = N_NODES // CHUNK
NBUF = 3

_INFO = plsc.get_sparse_core_info()
_NC = _INFO.num_cores
_NS = _INFO.num_subcores
_NW = _NC * _NS
_BASE = NCHUNK // _NW
_EXTRA = NCHUNK - _BASE * _NW


def _sc_body(x_hbm, idx_hbm, tab_hbm, out_hbm, *scratch):
    idx_v = scratch[0:NBUF]
    rows_v = scratch[NBUF:2 * NBUF]
    x_v = scratch[2 * NBUF:3 * NBUF]
    sem_g = scratch[3 * NBUF:4 * NBUF]
    sem_x = scratch[4 * NBUF:5 * NBUF]
    sem_o = scratch[5 * NBUF:6 * NBUF]

    wid = lax.axis_index("s") * _NC + lax.axis_index("c")
    first = wid * _BASE

    def load(c, b):
        base = c * CHUNK
        pltpu.sync_copy(idx_hbm.at[pl.ds(base, CHUNK)], idx_v[b])
        pltpu.async_copy(tab_hbm.at[idx_v[b]], rows_v[b], sem_g[b])
        pltpu.async_copy(x_hbm.at[pl.ds(base, CHUNK), :], x_v[b], sem_x[b])

    def wait_loads(b):
        pltpu.make_async_copy(tab_hbm.at[idx_v[b]], rows_v[b], sem_g[b]).wait()
        pltpu.make_async_copy(x_hbm.at[pl.ds(0, CHUNK), :], x_v[b],
                              sem_x[b]).wait()

    def add_rows(b):
        def row_body(r, carry):
            for c in range(D_FEAT // 16):
                sl = pl.ds(c * 16, 16)
                plsc.addupdate(x_v[b].at[r, sl], rows_v[b][r, sl])
            return carry
        lax.fori_loop(0, CHUNK, row_body, 0, unroll=False)

    def store(c, b):
        base = c * CHUNK
        pltpu.async_copy(x_v[b], out_hbm.at[pl.ds(base, CHUNK), :], sem_o[b])

    def wait_store(b):
        pltpu.make_async_copy(x_v[b], out_hbm.at[pl.ds(0, CHUNK), :],
                              sem_o[b]).wait()

    load(first, 0)

    def turn(j, carry):
        for b in range(NBUF):
            k = j * NBUF + b
            c = first + k
            bn = (b + 1) % NBUF
            @pl.when(k + 1 < _BASE)
            def _():
                @pl.when(k >= 2)
                def _():
                    wait_store(bn)
                load(c + 1, bn)
            wait_loads(b)
            add_rows(b)
            store(c, b)
        return carry

    lax.fori_loop(0, _BASE // NBUF, turn, 0, unroll=False)

    for b in range(NBUF):
        wait_store(b)

    @pl.when(wid < _EXTRA)
    def _():
        c = _NW * _BASE + wid
        load(c, 0)
        wait_loads(0)
        add_rows(0)
        store(c, 0)
        wait_store(0)


@jax.jit
def _run(x, idx, tab):
    mesh = plsc.VectorSubcoreMesh(core_axis_name="c", subcore_axis_name="s")
    f = pl.kernel(
        _sc_body,
        out_type=jax.ShapeDtypeStruct((N_NODES, D_FEAT), jnp.float32),
        mesh=mesh,
        scratch_types=(
            [pltpu.VMEM((CHUNK,), jnp.int32) for _ in range(NBUF)]
            + [pltpu.VMEM((CHUNK, D_FEAT), jnp.float32) for _ in range(NBUF)]
            + [pltpu.VMEM((CHUNK, D_FEAT), jnp.float32) for _ in range(NBUF)]
            + [pltpu.SemaphoreType.DMA for _ in range(3 * NBUF)]
        ),
    )
    return f(x, idx, tab)


def kernel(x, node_types, type_table):
    idx = node_types.astype(jnp.int32)
    return _run(x, idx, type_table)
